# Initial kernel scaffold; baseline (speedup 1.0000x reference)
import jax
import jax.numpy as jnp
from jax.experimental import pallas as pl


def kernel(x, edge_index, edge_attr, W1, b1, W2, b2, W3, b3, W4, b4):
    raise NotImplementedError("write your pallas kernel here")


# speedup vs baseline: 6.4637x; 6.4637x over previous
import functools

import jax
import jax.numpy as jnp
from jax import lax
from jax.experimental import pallas as pl
from jax.experimental.pallas import tpu as pltpu
from jax.experimental.pallas import tpu_sc as plsc

N = 10000
NP = 10240
E = 320000
NC, NS, L = 2, 16, 16
NW = NC * NS
C = 128
K = 79
EW = C * K
EP = EW * NW
RPS = NP // NS

@functools.cache
def _mesh():
    return plsc.VectorSubcoreMesh(
        core_axis_name="c", subcore_axis_name="s",
        num_cores=NC, num_subcores=NS,
    )


def _sc_agg_body(fo, h_hbm, src_hbm, dst_hbm, attr_hbm, out_hbm,
                 src_v, dst_v, attr_v, rows_v, acc_sh, sem):
    c = lax.axis_index("c")
    s = lax.axis_index("s")
    w = s * NC + c

    def _zero_rows(e, _):
        for q in range(fo // L):
            rows_v[e, pl.ds(q * L, L)] = jnp.zeros((L,), jnp.float32)
        return 0
    lax.fori_loop(0, C, _zero_rows, 0)
    for r in range(RPS // C):
        pltpu.sync_copy(rows_v, acc_sh.at[pl.ds(s * RPS + r * C, C)])
    plsc.subcore_barrier()

    def _chunk(g, _):
        base = w * EW + g * C
        pltpu.sync_copy(src_hbm.at[pl.ds(base, C)], src_v)
        pltpu.sync_copy(dst_hbm.at[pl.ds(base, C)], dst_v)
        pltpu.sync_copy(attr_hbm.at[pl.ds(base, C)], attr_v)
        pltpu.async_copy(h_hbm.at[src_v], rows_v, sem).wait()

        def _grp(j, _):
            ablk = attr_v[pl.ds(j * L, L)]
            for t in range(L):
                e = j * L + t
                sp = jnp.full((L,), ablk[t], jnp.float32)
                for q in range(fo // L):
                    rows_v[e, pl.ds(q * L, L)] = rows_v[e, pl.ds(q * L, L)] * sp
            return 0
        lax.fori_loop(0, C // L, _grp, 0)
        pltpu.sync_copy(rows_v, acc_sh.at[dst_v], add=True)
        return 0
    lax.fori_loop(0, K, _chunk, 0)
    plsc.subcore_barrier()

    for r in range(RPS // C):
        off = s * RPS + r * C
        pltpu.sync_copy(acc_sh.at[pl.ds(off, C)], out_hbm.at[c, pl.ds(off, C)])


@functools.cache
def _make_sc_agg(fo):
    return functools.partial(
        pl.kernel,
        out_type=jax.ShapeDtypeStruct((NC, NP, fo), jnp.float32),
        mesh=_mesh(),
        scratch_types=[
            pltpu.VMEM((C,), jnp.int32),
            pltpu.VMEM((C,), jnp.int32),
            pltpu.VMEM((C,), jnp.float32),
            pltpu.VMEM((C, fo), jnp.float32),
            pltpu.VMEM_SHARED((NP, fo), jnp.float32),
            pltpu.SemaphoreType.DMA,
        ],
        compiler_params=pltpu.CompilerParams(use_tc_tiling_on_sc=False),
    )(functools.partial(_sc_agg_body, fo))


def _sc_agg1_body(h_hbm, src_hbm, dst_hbm, attr_hbm, out_hbm,
                  src_v, dst_v, attr_v, rows_v, acc_sh, sem):
    c = lax.axis_index("c")
    s = lax.axis_index("s")
    w = s * NC + c

    def _zero(j, _):
        rows_v[pl.ds(j * L, L)] = jnp.zeros((L,), jnp.float32)
        return 0
    lax.fori_loop(0, C // L, _zero, 0)
    for r in range(RPS // C):
        pltpu.sync_copy(rows_v, acc_sh.at[pl.ds(s * RPS + r * C, C)])
    plsc.subcore_barrier()

    def _chunk(g, _):
        base = w * EW + g * C
        pltpu.sync_copy(src_hbm.at[pl.ds(base, C)], src_v)
        pltpu.sync_copy(dst_hbm.at[pl.ds(base, C)], dst_v)
        pltpu.sync_copy(attr_hbm.at[pl.ds(base, C)], attr_v)
        pltpu.async_copy(h_hbm.at[src_v], rows_v, sem).wait()
        for j in range(C // L):
            sl = pl.ds(j * L, L)
            rows_v[sl] = rows_v[sl] * attr_v[sl]
        pltpu.sync_copy(rows_v, acc_sh.at[dst_v], add=True)
        return 0
    lax.fori_loop(0, K, _chunk, 0)
    plsc.subcore_barrier()

    for r in range(RPS // C):
        off = s * RPS + r * C
        pltpu.sync_copy(acc_sh.at[pl.ds(off, C)], out_hbm.at[c, pl.ds(off, C)])


@functools.cache
def _make_agg1():
    return functools.partial(
        pl.kernel,
        out_type=jax.ShapeDtypeStruct((NC, NP), jnp.float32),
        mesh=_mesh(),
        scratch_types=[
            pltpu.VMEM((C,), jnp.int32),
            pltpu.VMEM((C,), jnp.int32),
            pltpu.VMEM((C,), jnp.float32),
            pltpu.VMEM((C,), jnp.float32),
            pltpu.VMEM_SHARED((NP,), jnp.float32),
            pltpu.SemaphoreType.DMA,
        ],
        compiler_params=pltpu.CompilerParams(use_tc_tiling_on_sc=False),
    )(_sc_agg1_body)


_BR = 1024


def _mm1_body(x_ref, w_ref, o_ref):
    o_ref[...] = jnp.dot(x_ref[...], w_ref[...],
                         preferred_element_type=jnp.float32)


def _mm1(x, W):
    fi, fo = W.shape
    return pl.pallas_call(
        _mm1_body,
        grid=(NP // _BR,),
        in_specs=[pl.BlockSpec((_BR, fi), lambda i: (i, 0)),
                  pl.BlockSpec((fi, fo), lambda i: (0, 0))],
        out_specs=pl.BlockSpec((_BR, fo), lambda i: (i, 0)),
        out_shape=jax.ShapeDtypeStruct((NP, fo), jnp.float32),
    )(x, W)


def _fused_body(p_ref, b_ref, w_ref, o_ref):
    h = jnp.maximum(p_ref[0] + p_ref[1] + b_ref[...], 0.0)
    o_ref[...] = jnp.dot(h, w_ref[...], preferred_element_type=jnp.float32)


def _fused(p, b, W):
    fi, fo = W.shape
    return pl.pallas_call(
        _fused_body,
        grid=(NP // _BR,),
        in_specs=[pl.BlockSpec((2, _BR, fi), lambda i: (0, i, 0)),
                  pl.BlockSpec((1, fi), lambda i: (0, 0)),
                  pl.BlockSpec((fi, fo), lambda i: (0, 0))],
        out_specs=pl.BlockSpec((_BR, fo), lambda i: (i, 0)),
        out_shape=jax.ShapeDtypeStruct((NP, fo), jnp.float32),
    )(p, b.reshape(1, fi), W)


def _last_body(p_ref, b_ref, w_ref, o_ref):
    h = jnp.maximum(p_ref[0] + p_ref[1] + b_ref[...], 0.0)
    o_ref[...] = jnp.sum(h * w_ref[...], axis=1, keepdims=True)


def _last(p, b, W):
    fi = W.shape[0]
    return pl.pallas_call(
        _last_body,
        grid=(NP // _BR,),
        in_specs=[pl.BlockSpec((2, _BR, fi), lambda i: (0, i, 0)),
                  pl.BlockSpec((1, fi), lambda i: (0, 0)),
                  pl.BlockSpec((1, fi), lambda i: (0, 0))],
        out_specs=pl.BlockSpec((_BR, 1), lambda i: (i, 0)),
        out_shape=jax.ShapeDtypeStruct((NP, 1), jnp.float32),
    )(p, b.reshape(1, fi), W.reshape(1, fi))


def _final_body(p_ref, b_ref, o_ref):
    z = p_ref[0] + p_ref[1] + b_ref[0, 0]
    o_ref[...] = jax.nn.sigmoid(z)


def _final(p, b):
    BC = 2048
    return pl.pallas_call(
        _final_body,
        grid=(NP // BC,),
        in_specs=[pl.BlockSpec((2, BC), lambda i: (0, i)),
                  pl.BlockSpec((1, 1), lambda i: (0, 0))],
        out_specs=pl.BlockSpec((BC,), lambda i: (i,)),
        out_shape=jax.ShapeDtypeStruct((NP,), jnp.float32),
    )(p, b.reshape(1, 1))


def kernel(x, edge_index, edge_attr, W1, b1, W2, b2, W3, b3, W4, b4):
    x_p = jnp.zeros((NP, x.shape[1]), jnp.float32).at[:N].set(x)
    pad = EP - E
    src_p = jnp.concatenate([edge_index[0], jnp.zeros((pad,), jnp.int32)])
    dst_p = jnp.concatenate([edge_index[1], jnp.zeros((pad,), jnp.int32)])
    attr_p = jnp.concatenate([edge_attr, jnp.zeros((pad,), jnp.float32)])

    h1 = _mm1(x_p, W1)
    p1 = _make_sc_agg(64)(h1, src_p, dst_p, attr_p)
    h2 = _fused(p1, b1, W2)
    p2 = _make_sc_agg(32)(h2, src_p, dst_p, attr_p)
    h3 = _fused(p2, b2, W3)
    p3 = _make_sc_agg(16)(h3, src_p, dst_p, attr_p)
    h4 = _last(p3, b3, W4)
    p4 = _make_agg1()(h4.reshape(NP), src_p, dst_p, attr_p)
    out = _final(p4, b4)
    return out[:N, None]


# speedup vs baseline: 11.0908x; 1.7159x over previous
import functools

import jax
import jax.numpy as jnp
from jax import lax
from jax.experimental import pallas as pl
from jax.experimental.pallas import tpu as pltpu
from jax.experimental.pallas import tpu_sc as plsc

N = 10000
NP = 10240
E = 320000
NC, NS, L = 2, 16, 16
NW = NC * NS
C = 128
K = 80
EW = C * K
EP = EW * NW
RPS = NP // NS
NB = 4
D = NB - 1


@functools.cache
def _mesh():
    return plsc.VectorSubcoreMesh(
        core_axis_name="c", subcore_axis_name="s",
        num_cores=NC, num_subcores=NS,
    )


def _sc_agg_body(fo, h_hbm, src_hbm, dst_hbm, attr_hbm, out_hbm,
                 src_v, dst_v, attr_v, b0, b1, b2, b3, acc_sh,
                 g0, g1, g2, g3, s0, s1, s2, s3):
    bufs = (b0, b1, b2, b3)
    gsem = (g0, g1, g2, g3)
    ssem = (s0, s1, s2, s3)
    c = lax.axis_index("c")
    s = lax.axis_index("s")
    w = s * NC + c

    pltpu.sync_copy(src_hbm.at[w], src_v)
    pltpu.sync_copy(dst_hbm.at[w], dst_v)
    pltpu.sync_copy(attr_hbm.at[w], attr_v)

    if fo == 1:
        def _zero(j, _):
            b3[pl.ds(j * L, L)] = jnp.zeros((L,), jnp.float32)
            return 0
        lax.fori_loop(0, C // L, _zero, 0)
    else:
        def _zero(e, _):
            for q in range(fo // L):
                b3[e, pl.ds(q * L, L)] = jnp.zeros((L,), jnp.float32)
            return 0
        lax.fori_loop(0, C, _zero, 0)
    for r in range(RPS // C):
        pltpu.async_copy(b3, acc_sh.at[pl.ds(s * RPS + r * C, C)], g3)
    for r in range(RPS // C):
        pltpu.make_async_copy(b3, acc_sh.at[pl.ds(s * RPS + r * C, C)], g3).wait()

    for g in range(D):
        pltpu.async_copy(h_hbm.at[src_v.at[g]], bufs[g], gsem[g])
    plsc.subcore_barrier()

    def _group(grp, _):
        G = grp * NB
        for b in range(NB):
            g = G + b
            pb = (b - 1) % NB

            @pl.when(jnp.logical_and(g >= 1, g + D < K))
            def _():
                pltpu.make_async_copy(
                    bufs[pb], acc_sh.at[dst_v.at[g - 1]], ssem[pb]).wait()

            @pl.when(g + D < K)
            def _():
                pltpu.async_copy(
                    h_hbm.at[src_v.at[g + D]], bufs[pb], gsem[pb])

            pltpu.make_async_copy(
                h_hbm.at[src_v.at[g]], bufs[b], gsem[b]).wait()

            if fo == 1:
                for j in range(C // L):
                    sl = pl.ds(j * L, L)
                    bufs[b][sl] = bufs[b][sl] * attr_v[g, sl]
            else:
                def _mul(j, _, b=b, g=g):
                    ablk = attr_v[g, pl.ds(j * L, L)]
                    for t in range(L):
                        e = j * L + t
                        sp = jnp.full((L,), ablk[t], jnp.float32)
                        for q in range(fo // L):
                            sl = pl.ds(q * L, L)
                            bufs[b][e, sl] = bufs[b][e, sl] * sp
                    return 0
                lax.fori_loop(0, C // L, _mul, 0)

            pltpu.async_copy(
                bufs[b], acc_sh.at[dst_v.at[g]], ssem[b], add=True)
        return 0
    lax.fori_loop(0, K // NB, _group, 0)

    for b in range(NB):
        gl = K - NB + b
        pltpu.make_async_copy(bufs[b], acc_sh.at[dst_v.at[gl]], ssem[b]).wait()
    plsc.subcore_barrier()

    pltpu.sync_copy(acc_sh.at[pl.ds(s * RPS, RPS)],
                    out_hbm.at[c, pl.ds(s * RPS, RPS)])


def _sc_scratch(fo):
    if fo == 1:
        buf = lambda: pltpu.VMEM((C,), jnp.float32)
        acc = pltpu.VMEM_SHARED((NP,), jnp.float32)
    else:
        buf = lambda: pltpu.VMEM((C, fo), jnp.float32)
        acc = pltpu.VMEM_SHARED((NP, fo), jnp.float32)
    return [
        pltpu.VMEM((K, C), jnp.int32),
        pltpu.VMEM((K, C), jnp.int32),
        pltpu.VMEM((K, C), jnp.float32),
        buf(), buf(), buf(), buf(),
        acc,
    ] + [pltpu.SemaphoreType.DMA] * (2 * NB)


@functools.cache
def _make_sc_agg(fo):
    out_shape = (NC, NP) if fo == 1 else (NC, NP, fo)
    return functools.partial(
        pl.kernel,
        out_type=jax.ShapeDtypeStruct(out_shape, jnp.float32),
        mesh=_mesh(),
        scratch_types=_sc_scratch(fo),
        compiler_params=pltpu.CompilerParams(use_tc_tiling_on_sc=False),
    )(functools.partial(_sc_agg_body, fo))


_BR = 1024


def _mm1_body(x_ref, w_ref, o_ref):
    o_ref[...] = jnp.dot(x_ref[...], w_ref[...],
                         preferred_element_type=jnp.float32)


def _mm1(x, W):
    fi, fo = W.shape
    return pl.pallas_call(
        _mm1_body,
        grid=(NP // _BR,),
        in_specs=[pl.BlockSpec((_BR, fi), lambda i: (i, 0)),
                  pl.BlockSpec((fi, fo), lambda i: (0, 0))],
        out_specs=pl.BlockSpec((_BR, fo), lambda i: (i, 0)),
        out_shape=jax.ShapeDtypeStruct((NP, fo), jnp.float32),
    )(x, W)


def _fused_body(p_ref, b_ref, w_ref, o_ref):
    h = jnp.maximum(p_ref[0] + p_ref[1] + b_ref[...], 0.0)
    o_ref[...] = jnp.dot(h, w_ref[...], preferred_element_type=jnp.float32)


def _fused(p, b, W):
    fi, fo = W.shape
    return pl.pallas_call(
        _fused_body,
        grid=(NP // _BR,),
        in_specs=[pl.BlockSpec((2, _BR, fi), lambda i: (0, i, 0)),
                  pl.BlockSpec((1, fi), lambda i: (0, 0)),
                  pl.BlockSpec((fi, fo), lambda i: (0, 0))],
        out_specs=pl.BlockSpec((_BR, fo), lambda i: (i, 0)),
        out_shape=jax.ShapeDtypeStruct((NP, fo), jnp.float32),
    )(p, b.reshape(1, fi), W)


def _last_body(p_ref, b_ref, w_ref, o_ref):
    h = jnp.maximum(p_ref[0] + p_ref[1] + b_ref[...], 0.0)
    o_ref[...] = jnp.sum(h * w_ref[...], axis=1, keepdims=True)


def _last(p, b, W):
    fi = W.shape[0]
    return pl.pallas_call(
        _last_body,
        grid=(NP // _BR,),
        in_specs=[pl.BlockSpec((2, _BR, fi), lambda i: (0, i, 0)),
                  pl.BlockSpec((1, fi), lambda i: (0, 0)),
                  pl.BlockSpec((1, fi), lambda i: (0, 0))],
        out_specs=pl.BlockSpec((_BR, 1), lambda i: (i, 0)),
        out_shape=jax.ShapeDtypeStruct((NP, 1), jnp.float32),
    )(p, b.reshape(1, fi), W.reshape(1, fi))


def _final_body(p_ref, b_ref, o_ref):
    z = p_ref[0] + p_ref[1] + b_ref[0, 0]
    o_ref[...] = jax.nn.sigmoid(z)


def _final(p, b):
    BC = 2048
    return pl.pallas_call(
        _final_body,
        grid=(NP // BC,),
        in_specs=[pl.BlockSpec((2, BC), lambda i: (0, i)),
                  pl.BlockSpec((1, 1), lambda i: (0, 0))],
        out_specs=pl.BlockSpec((BC,), lambda i: (i,)),
        out_shape=jax.ShapeDtypeStruct((NP,), jnp.float32),
    )(p, b.reshape(1, 1))


def kernel(x, edge_index, edge_attr, W1, b1, W2, b2, W3, b3, W4, b4):
    x_p = jnp.zeros((NP, x.shape[1]), jnp.float32).at[:N].set(x)
    pad = EP - E
    src_p = jnp.concatenate(
        [edge_index[0], jnp.zeros((pad,), jnp.int32)]).reshape(NW, K, C)
    dst_p = jnp.concatenate(
        [edge_index[1], jnp.zeros((pad,), jnp.int32)]).reshape(NW, K, C)
    attr_p = jnp.concatenate(
        [edge_attr, jnp.zeros((pad,), jnp.float32)]).reshape(NW, K, C)

    h1 = _mm1(x_p, W1)
    p1 = _make_sc_agg(64)(h1, src_p, dst_p, attr_p)
    h2 = _fused(p1, b1, W2)
    p2 = _make_sc_agg(32)(h2, src_p, dst_p, attr_p)
    h3 = _fused(p2, b2, W3)
    p3 = _make_sc_agg(16)(h3, src_p, dst_p, attr_p)
    h4 = _last(p3, b3, W4)
    p4 = _make_sc_agg(1)(h4.reshape(NP), src_p, dst_p, attr_p)
    out = _final(p4, b4)
    return out[:N, None]
